# baseline (speedup 1.0000x reference)
import jax
import jax.numpy as jnp
from jax import lax
from jax.experimental import pallas as pl
from jax.experimental.pallas import tpu as pltpu

_B = 4
_N = 2048
_NFEAT = 128
_NHID = 64
_NHEADS = 4
_OUTC = 8
_ROWS = 1024
_SLOPE = 0.2

_NT = (((1,), (1,)), ((), ()))
_HI = lax.Precision.HIGHEST
_BF = jnp.bfloat16


def _leaky(v):
    return jnp.where(v >= 0, v, _SLOPE * v)


def _elu(v):
    return jnp.where(v > 0, v, jnp.exp(jnp.minimum(v, 0.0)) - 1.0)


def _rtne_bf16(v):
    u = lax.bitcast_convert_type(v, jnp.uint32)
    bias = jnp.uint32(0x7FFF) + ((u >> jnp.uint32(16)) & jnp.uint32(1))
    u2 = (u + bias) & jnp.uint32(0xFFFF0000)
    return lax.bitcast_convert_type(u2, jnp.float32)


def _hilo(v):
    hi = _rtne_bf16(v)
    lo = _rtne_bf16(v - hi)
    return hi.astype(_BF), lo.astype(_BF)


def _att_tables(Wh, t_col):
    tmax = jnp.max(t_col)
    w1 = jnp.exp(t_col - tmax)
    w2 = jnp.exp(_SLOPE * (t_col - tmax))
    r1h, r1l = _hilo(w1 * Wh)
    r2h, r2l = _hilo(w2 * Wh)
    zc = jnp.concatenate([w1, w2], axis=1)
    zch, zcl = _hilo(zc)
    rhs = jnp.concatenate([r1h, r1l, r2h, r2l, zch, zcl], axis=1)
    t2row = jnp.sum(w2 * Wh, axis=0, keepdims=True)
    tot2 = jnp.sum(w2)
    return rhs, t2row, tot2, tmax


def _att_chunk(sc, tT, rhs, t2row, tot2, tmax, width):
    f32 = jnp.float32
    mb = ((sc + tT) > 0).astype(_BF)
    half = _N // 2
    p = (jnp.dot(mb[:, :half], rhs[:half], preferred_element_type=f32)
         + jnp.dot(mb[:, half:], rhs[half:], preferred_element_type=f32))
    w = width
    A = p[:, 0:w] + p[:, w:2 * w]
    Bt = p[:, 2 * w:3 * w] + p[:, 3 * w:4 * w]
    z1 = p[:, 4 * w:4 * w + 1] + p[:, 4 * w + 2:4 * w + 3]
    z2t = p[:, 4 * w + 1:4 * w + 2] + p[:, 4 * w + 3:4 * w + 4]
    spt = sc + tmax
    m = _leaky(spt)
    cA = jnp.exp(spt - m)
    cB = jnp.exp(_SLOPE * spt - m)
    num = cA * A + cB * (t2row - Bt)
    den = cA * z1 + cB * (tot2 - z2t)
    return num / den


def _gat_body(x_ref, wh_ref, ah_ref, wo_ref, ao_ref, wl_ref, out_ref, h_scr):
    f32 = jnp.float32
    xb = x_ref[0]

    for k in range(_NHEADS):
        Wk = wh_ref[k]
        Wh = jnp.dot(xb, Wk, preferred_element_type=f32, precision=_HI)
        a1 = ah_ref[k:k + 1, 0:_NHID]
        a2 = ah_ref[k:k + 1, _NHID:]
        s_col = jnp.sum(Wh * a1, axis=1, keepdims=True)
        t_col = jnp.sum(Wh * a2, axis=1, keepdims=True)
        tT = lax.dot_general(a2, Wh, _NT, preferred_element_type=f32,
                             precision=_HI)
        rhs, t2row, tot2, tmax = _att_tables(Wh, t_col)
        for c in range(_N // _ROWS):
            sc = s_col[c * _ROWS:(c + 1) * _ROWS]
            hc = _att_chunk(sc, tT, rhs, t2row, tot2, tmax, _NHID)
            h_scr[c * _ROWS:(c + 1) * _ROWS, k * _NHID:(k + 1) * _NHID] = _elu(hc)

    h = h_scr[:, :]
    Wh2 = jnp.dot(h, wo_ref[:, :], preferred_element_type=f32, precision=_HI)
    a1o = ao_ref[0:1, 0:_OUTC]
    a2o = ao_ref[0:1, _OUTC:]
    s2_col = jnp.sum(Wh2 * a1o, axis=1, keepdims=True)
    t2_col = jnp.sum(Wh2 * a2o, axis=1, keepdims=True)
    t2T = lax.dot_general(a2o, Wh2, _NT, preferred_element_type=f32,
                          precision=_HI)
    rhs2, t2row2, tot22, tmax2 = _att_tables(Wh2, t2_col)
    acc = jnp.float32(0.0)
    for c in range(_N // _ROWS):
        sc = s2_col[c * _ROWS:(c + 1) * _ROWS]
        hc = _att_chunk(sc, t2T, rhs2, t2row2, tot22, tmax2, _OUTC)
        acc = acc + jnp.sum(_elu(hc) * wl_ref[c * _ROWS:(c + 1) * _ROWS, :])
    out_ref[0] = jnp.full((8, 128), acc, dtype=f32)


def kernel(x, W_heads, a_heads, W_out, a_out, W_lin, b_lin):
    ah = a_heads.reshape(_NHEADS, 2 * _NHID)
    ao = a_out.reshape(1, 2 * _OUTC)
    wl = W_lin.reshape(_N, _OUTC)
    out = pl.pallas_call(
        _gat_body,
        grid=(_B,),
        in_specs=[
            pl.BlockSpec((1, _N, _NFEAT), lambda b: (b, 0, 0)),
            pl.BlockSpec((_NHEADS, _NFEAT, _NHID), lambda b: (0, 0, 0)),
            pl.BlockSpec((_NHEADS, 2 * _NHID), lambda b: (0, 0)),
            pl.BlockSpec((_NHEADS * _NHID, _OUTC), lambda b: (0, 0)),
            pl.BlockSpec((1, 2 * _OUTC), lambda b: (0, 0)),
            pl.BlockSpec((_N, _OUTC), lambda b: (0, 0)),
        ],
        out_specs=pl.BlockSpec((1, 8, 128), lambda b: (b, 0, 0)),
        out_shape=jax.ShapeDtypeStruct((_B, 8, 128), jnp.float32),
        scratch_shapes=[pltpu.VMEM((_N, _NHEADS * _NHID), jnp.float32)],
    )(x, W_heads, ah, W_out, ao, wl)
    return out[:, 0, :1] + b_lin


# speedup vs baseline: 1.0032x; 1.0032x over previous
import jax
import jax.numpy as jnp
from jax import lax
from jax.experimental import pallas as pl
from jax.experimental.pallas import tpu as pltpu

_B = 4
_N = 2048
_NFEAT = 128
_NHID = 64
_NHEADS = 4
_OUTC = 8
_ROWS = 1024
_SLOPE = 0.2

_NT = (((1,), (1,)), ((), ()))
_HI = lax.Precision.HIGHEST
_BF = jnp.bfloat16


def _leaky(v):
    return jnp.where(v >= 0, v, _SLOPE * v)


def _elu(v):
    return jnp.where(v > 0, v, jnp.exp(jnp.minimum(v, 0.0)) - 1.0)


def _rtne_bf16(v):
    u = lax.bitcast_convert_type(v, jnp.uint32)
    bias = jnp.uint32(0x7FFF) + ((u >> jnp.uint32(16)) & jnp.uint32(1))
    u2 = (u + bias) & jnp.uint32(0xFFFF0000)
    return lax.bitcast_convert_type(u2, jnp.float32)


def _hilo(v):
    hi = _rtne_bf16(v)
    lo = _rtne_bf16(v - hi)
    return hi.astype(_BF), lo.astype(_BF)


def _att_tables(Wh, t_col, tT):
    tmax = jnp.max(t_col)
    w1 = jnp.exp(t_col - tmax)
    w2 = jnp.exp(_SLOPE * (t_col - tmax))
    r1h, r1l = _hilo(w1 * Wh)
    r2h, r2l = _hilo(w2 * Wh)
    rhs = jnp.concatenate([r1h, r1l, r2h, r2l], axis=1)
    w1row = jnp.exp(tT - tmax)
    w2row = jnp.exp(_SLOPE * (tT - tmax))
    t2row = jnp.sum(w2 * Wh, axis=0, keepdims=True)
    tot2 = jnp.sum(w2row)
    return rhs, t2row, tot2, tmax, w1row, w2row


def _att_chunk(sc, tT, rhs, t2row, tot2, tmax, w1row, w2row, width):
    f32 = jnp.float32
    cond = (sc + tT) > 0
    mb = cond.astype(_BF)
    z1 = jnp.sum(jnp.where(cond, w1row, 0.0), axis=1, keepdims=True)
    z2t = jnp.sum(jnp.where(cond, w2row, 0.0), axis=1, keepdims=True)
    half = _N // 2
    p = (jnp.dot(mb[:, :half], rhs[:half], preferred_element_type=f32)
         + jnp.dot(mb[:, half:], rhs[half:], preferred_element_type=f32))
    w = width
    A = p[:, 0:w] + p[:, w:2 * w]
    Bt = p[:, 2 * w:3 * w] + p[:, 3 * w:4 * w]
    spt = sc + tmax
    m = _leaky(spt)
    cA = jnp.exp(spt - m)
    cB = jnp.exp(_SLOPE * spt - m)
    num = cA * A + cB * (t2row - Bt)
    den = cA * z1 + cB * (tot2 - z2t)
    return num / den


def _gat_body(x_ref, wh_ref, ah_ref, wo_ref, ao_ref, wl_ref, out_ref, h_scr):
    f32 = jnp.float32
    xb = x_ref[0]

    for k in range(_NHEADS):
        Wk = wh_ref[k]
        Wh = jnp.dot(xb, Wk, preferred_element_type=f32, precision=_HI)
        a1 = ah_ref[k:k + 1, 0:_NHID]
        a2 = ah_ref[k:k + 1, _NHID:]
        s_col = jnp.sum(Wh * a1, axis=1, keepdims=True)
        t_col = jnp.sum(Wh * a2, axis=1, keepdims=True)
        tT = lax.dot_general(a2, Wh, _NT, preferred_element_type=f32,
                             precision=_HI)
        rhs, t2row, tot2, tmax, w1r, w2r = _att_tables(Wh, t_col, tT)
        for c in range(_N // _ROWS):
            sc = s_col[c * _ROWS:(c + 1) * _ROWS]
            hc = _att_chunk(sc, tT, rhs, t2row, tot2, tmax, w1r, w2r, _NHID)
            h_scr[c * _ROWS:(c + 1) * _ROWS, k * _NHID:(k + 1) * _NHID] = _elu(hc)

    h = h_scr[:, :]
    Wh2 = jnp.dot(h, wo_ref[:, :], preferred_element_type=f32, precision=_HI)
    a1o = ao_ref[0:1, 0:_OUTC]
    a2o = ao_ref[0:1, _OUTC:]
    s2_col = jnp.sum(Wh2 * a1o, axis=1, keepdims=True)
    t2_col = jnp.sum(Wh2 * a2o, axis=1, keepdims=True)
    t2T = lax.dot_general(a2o, Wh2, _NT, preferred_element_type=f32,
                          precision=_HI)
    rhs2, t2row2, tot22, tmax2, w1r2, w2r2 = _att_tables(Wh2, t2_col, t2T)
    acc = jnp.float32(0.0)
    for c in range(_N // _ROWS):
        sc = s2_col[c * _ROWS:(c + 1) * _ROWS]
        hc = _att_chunk(sc, t2T, rhs2, t2row2, tot22, tmax2, w1r2, w2r2, _OUTC)
        acc = acc + jnp.sum(_elu(hc) * wl_ref[c * _ROWS:(c + 1) * _ROWS, :])
    out_ref[0] = jnp.full((8, 128), acc, dtype=f32)


def kernel(x, W_heads, a_heads, W_out, a_out, W_lin, b_lin):
    ah = a_heads.reshape(_NHEADS, 2 * _NHID)
    ao = a_out.reshape(1, 2 * _OUTC)
    wl = W_lin.reshape(_N, _OUTC)
    out = pl.pallas_call(
        _gat_body,
        grid=(_B,),
        in_specs=[
            pl.BlockSpec((1, _N, _NFEAT), lambda b: (b, 0, 0)),
            pl.BlockSpec((_NHEADS, _NFEAT, _NHID), lambda b: (0, 0, 0)),
            pl.BlockSpec((_NHEADS, 2 * _NHID), lambda b: (0, 0)),
            pl.BlockSpec((_NHEADS * _NHID, _OUTC), lambda b: (0, 0)),
            pl.BlockSpec((1, 2 * _OUTC), lambda b: (0, 0)),
            pl.BlockSpec((_N, _OUTC), lambda b: (0, 0)),
        ],
        out_specs=pl.BlockSpec((1, 8, 128), lambda b: (b, 0, 0)),
        out_shape=jax.ShapeDtypeStruct((_B, 8, 128), jnp.float32),
        scratch_shapes=[pltpu.VMEM((_N, _NHEADS * _NHID), jnp.float32)],
    )(x, W_heads, ah, W_out, ao, wl)
    return out[:, 0, :1] + b_lin


# speedup vs baseline: 1.0143x; 1.0111x over previous
import jax
import jax.numpy as jnp
from jax import lax
from jax.experimental import pallas as pl
from jax.experimental.pallas import tpu as pltpu

_B = 4
_N = 2048
_NFEAT = 128
_NHID = 64
_NHEADS = 4
_OUTC = 8
_ROWS = 1024
_SLOPE = 0.2

_NT = (((1,), (1,)), ((), ()))
_HI = lax.Precision.HIGHEST
_BF = jnp.bfloat16


def _leaky(v):
    return jnp.where(v >= 0, v, _SLOPE * v)


def _elu(v):
    return jnp.where(v > 0, v, jnp.exp(jnp.minimum(v, 0.0)) - 1.0)


def _rtne_bf16(v):
    u = lax.bitcast_convert_type(v, jnp.uint32)
    bias = jnp.uint32(0x7FFF) + ((u >> jnp.uint32(16)) & jnp.uint32(1))
    u2 = (u + bias) & jnp.uint32(0xFFFF0000)
    return lax.bitcast_convert_type(u2, jnp.float32)


def _hilo(v):
    hi = _rtne_bf16(v)
    lo = _rtne_bf16(v - hi)
    return hi.astype(_BF), lo.astype(_BF)


def _att_tables(Wh, t_col, tT):
    tmax = jnp.max(t_col)
    w1 = jnp.exp(t_col - tmax)
    w2 = jnp.exp(_SLOPE * (t_col - tmax))
    r1h, r1l = _hilo(w1 * Wh)
    r2h, r2l = _hilo(w2 * Wh)
    rhs = jnp.concatenate([r1h, r1l, r2h, r2l], axis=1)
    w1row = jnp.exp(tT - tmax)
    w2row = jnp.exp(_SLOPE * (tT - tmax))
    t2row = jnp.sum(w2 * Wh, axis=0, keepdims=True)
    tot2 = jnp.sum(w2row)
    return rhs, t2row, tot2, tmax, w1row, w2row


def _att_chunk(sc, tT, rhs, t2row, tot2, tmax, w1row, w2row, width):
    f32 = jnp.float32
    cond = (sc + tT) > 0
    mb = cond.astype(_BF)
    z1 = jnp.sum(jnp.where(cond, w1row, 0.0), axis=1, keepdims=True)
    z2t = jnp.sum(jnp.where(cond, w2row, 0.0), axis=1, keepdims=True)
    half = _N // 2
    p = (jnp.dot(mb[:, :half], rhs[:half], preferred_element_type=f32)
         + jnp.dot(mb[:, half:], rhs[half:], preferred_element_type=f32))
    w = width
    A = p[:, 0:w] + p[:, w:2 * w]
    Bt = p[:, 2 * w:3 * w] + p[:, 3 * w:4 * w]
    spt = sc + tmax
    m = _leaky(spt)
    cA = jnp.exp(spt - m)
    cB = jnp.exp(_SLOPE * spt - m)
    num = cA * A + cB * (t2row - Bt)
    den = cA * z1 + cB * (tot2 - z2t)
    return num / den


def _gat_body(x_ref, wh_ref, ah_ref, wo_ref, ao_ref, wl_ref, out_ref, h_scr):
    f32 = jnp.float32
    xb = x_ref[0]

    for k in range(_NHEADS):
        Wk = wh_ref[k]
        Wh = jnp.dot(xb, Wk, preferred_element_type=f32, precision=_HI)
        a1 = ah_ref[k:k + 1, 0:_NHID]
        a2 = ah_ref[k:k + 1, _NHID:]
        s_col = jnp.sum(Wh * a1, axis=1, keepdims=True)
        t_col = jnp.sum(Wh * a2, axis=1, keepdims=True)
        tT = lax.dot_general(a2, Wh, _NT, preferred_element_type=f32,
                             precision=_HI)
        rhs, t2row, tot2, tmax, w1r, w2r = _att_tables(Wh, t_col, tT)
        for c in range(_N // _ROWS):
            sc = s_col[c * _ROWS:(c + 1) * _ROWS]
            hc = _att_chunk(sc, tT, rhs, t2row, tot2, tmax, w1r, w2r, _NHID)
            h_scr[c * _ROWS:(c + 1) * _ROWS, k * _NHID:(k + 1) * _NHID] = _elu(hc)

    h = h_scr[:, :]
    Wh2 = jnp.dot(h, wo_ref[:, :], preferred_element_type=f32, precision=_HI)
    a1o = ao_ref[0:1, 0:_OUTC]
    a2o = ao_ref[0:1, _OUTC:]
    s2_col = jnp.sum(Wh2 * a1o, axis=1, keepdims=True)
    t2_col = jnp.sum(Wh2 * a2o, axis=1, keepdims=True)
    t2T = lax.dot_general(a2o, Wh2, _NT, preferred_element_type=f32,
                          precision=_HI)
    rhs2, t2row2, tot22, tmax2, w1r2, w2r2 = _att_tables(Wh2, t2_col, t2T)
    acc = jnp.float32(0.0)
    for c in range(_N // _ROWS):
        sc = s2_col[c * _ROWS:(c + 1) * _ROWS]
        hc = _att_chunk(sc, t2T, rhs2, t2row2, tot22, tmax2, w1r2, w2r2, _OUTC)
        acc = acc + jnp.sum(_elu(hc) * wl_ref[c * _ROWS:(c + 1) * _ROWS, :])
    out_ref[0] = jnp.full((8, 128), acc, dtype=f32)


def kernel(x, W_heads, a_heads, W_out, a_out, W_lin, b_lin):
    ah = a_heads.reshape(_NHEADS, 2 * _NHID)
    ao = a_out.reshape(1, 2 * _OUTC)
    wl = W_lin.reshape(_N, _OUTC)
    out = pl.pallas_call(
        _gat_body,
        grid=(_B,),
        in_specs=[
            pl.BlockSpec((1, _N, _NFEAT), lambda b: (b, 0, 0)),
            pl.BlockSpec((_NHEADS, _NFEAT, _NHID), lambda b: (0, 0, 0)),
            pl.BlockSpec((_NHEADS, 2 * _NHID), lambda b: (0, 0)),
            pl.BlockSpec((_NHEADS * _NHID, _OUTC), lambda b: (0, 0)),
            pl.BlockSpec((1, 2 * _OUTC), lambda b: (0, 0)),
            pl.BlockSpec((_N, _OUTC), lambda b: (0, 0)),
        ],
        out_specs=pl.BlockSpec((1, 8, 128), lambda b: (b, 0, 0)),
        out_shape=jax.ShapeDtypeStruct((_B, 8, 128), jnp.float32),
        scratch_shapes=[pltpu.VMEM((_N, _NHEADS * _NHID), jnp.float32)],
        compiler_params=pltpu.CompilerParams(
            dimension_semantics=("parallel",)),
    )(x, W_heads, ah, W_out, ao, wl)
    return out[:, 0, :1] + b_lin
